# baseline (speedup 1.0000x reference)
import jax
import jax.numpy as jnp
from jax import lax
from jax.experimental import pallas as pl
from jax.experimental.pallas import tpu as pltpu
from jax.experimental.pallas import tpu_sc as plsc

N = 10000
E = 320000
IN_DIM = 128
HID = 64
ZD = 32

NC, NS, L = 2, 16, 16
NW = NC * NS
EP = E // NW
EPD = 2 * E // NW
SUBK = 80
NSUBC = 5
CHE = SUBK * NSUBC
NCH_A = EP // CHE
NCH_D = EPD // CHE
NPAD = 10240
NPT = NPAD // NS
ZROWS = 160

RB = 1000
GRID = N // RB

_MESH = plsc.VectorSubcoreMesh(
    core_axis_name="c", subcore_axis_name="s", num_cores=NC, num_subcores=NS)
_SC_PARAMS = pltpu.CompilerParams(needs_layout_passes=False,
                                  use_tc_tiling_on_sc=False)


def _sc_deg_body(dst_hbm, out_hbm, dstbuf, countbuf):
    cid = lax.axis_index("c")
    sid = lax.axis_index("s")
    wid = sid * NC + cid
    zeros16 = jnp.zeros((L,), jnp.float32)
    ones16 = jnp.ones((L,), jnp.float32)

    def zb(i, c):
        countbuf[pl.ds(i * L, L)] = zeros16
        return c
    lax.fori_loop(0, N // L, zb, 0)

    pltpu.sync_copy(dst_hbm.at[pl.ds(wid * EP, EP)], dstbuf)

    def cb(i, c):
        idx = dstbuf[pl.ds(i * L, L)]
        plsc.addupdate_scatter(countbuf, [idx], ones16)
        return c
    lax.fori_loop(0, EP // L, cb, 0)

    for g in range(GRID):
        pltpu.sync_copy(countbuf.at[pl.ds(g * RB, RB)], out_hbm.at[g, wid])


_deg_call = pl.kernel(
    _sc_deg_body,
    out_type=jax.ShapeDtypeStruct((GRID, NW, RB), jnp.float32),
    mesh=_MESH,
    compiler_params=_SC_PARAMS,
    scratch_types=[
        pltpu.VMEM((EP,), jnp.int32),
        pltpu.VMEM((N,), jnp.float32),
    ],
)


def _sc_agg_body(tab_hbm, src_hbm, dst_hbm, out_hbm, sidx, didx, rows, zbuf,
                 acc, gsem, ssem):
    cid = lax.axis_index("c")
    sid = lax.axis_index("s")
    wid = sid * NC + cid
    zeros16 = jnp.zeros((L,), jnp.float32)

    def zb(i, c):
        for j in range(HID // L):
            zbuf[i, pl.ds(j * L, L)] = zeros16
        return c
    lax.fori_loop(0, ZROWS, zb, 0)
    for r in range(NPT // ZROWS):
        pltpu.sync_copy(zbuf, acc.at[pl.ds(sid * NPT + r * ZROWS, ZROWS)])
    plsc.subcore_barrier()

    pltpu.sync_copy(src_hbm.at[pl.ds(wid * EP, EP)], sidx)
    pltpu.sync_copy(dst_hbm.at[pl.ds(wid * EP, EP)], didx)

    def start_gathers(c, b):
        for j in range(NSUBC):
            pltpu.async_copy(
                tab_hbm.at[sidx.at[pl.ds(c * CHE + j * SUBK, SUBK)]],
                rows.at[b].at[pl.ds(j * SUBK, SUBK)], gsem.at[b])

    start_gathers(0, 0)
    start_gathers(1, 1)

    def body(c, carry):
        b = lax.rem(c, 2)
        pltpu.make_async_copy(tab_hbm.at[pl.ds(0, CHE)], rows.at[b],
                              gsem.at[b]).wait()
        for j in range(NSUBC):
            pltpu.async_copy(rows.at[b].at[pl.ds(j * SUBK, SUBK)],
                             acc.at[didx.at[pl.ds((c * NSUBC + j) * SUBK, SUBK)]],
                             ssem.at[b], add=True)
        pltpu.make_async_copy(tab_hbm.at[pl.ds(0, CHE)], rows.at[b],
                              ssem.at[b]).wait()

        @pl.when(c + 2 < NCH_A)
        def _():
            start_gathers(c + 2, b)
        return carry
    lax.fori_loop(0, NCH_A, body, 0)

    plsc.subcore_barrier()
    pltpu.sync_copy(acc.at[pl.ds(sid * NPT, NPT)],
                    out_hbm.at[cid, pl.ds(sid * NPT, NPT)])


_agg_call = pl.kernel(
    _sc_agg_body,
    out_type=jax.ShapeDtypeStruct((NC, NPAD, HID), jnp.float32),
    mesh=_MESH,
    compiler_params=_SC_PARAMS,
    scratch_types=[
        pltpu.VMEM((EP,), jnp.int32),
        pltpu.VMEM((EP,), jnp.int32),
        pltpu.VMEM((2, CHE, HID), jnp.float32),
        pltpu.VMEM((ZROWS, HID), jnp.float32),
        pltpu.VMEM_SHARED((NPAD, HID), jnp.float32),
        pltpu.SemaphoreType.DMA((2,)),
        pltpu.SemaphoreType.DMA((2,)),
    ],
)


PTP = 24


def _sc_dec_body(z_hbm, srcp_hbm, dstp_hbm, srcn_hbm, dstn_hbm, out_hbm,
                 sidx, didx, zs, zd, lbuf, ptmp, gsem):
    cid = lax.axis_index("c")
    sid = lax.axis_index("s")
    wid = sid * NC + cid
    iota = lax.iota(jnp.int32, L)
    zeros16 = jnp.zeros((L,), jnp.float32)

    pltpu.sync_copy(srcp_hbm.at[pl.ds(wid * EP, EP)], sidx.at[pl.ds(0, EP)])
    pltpu.sync_copy(srcn_hbm.at[pl.ds(wid * EP, EP)], sidx.at[pl.ds(EP, EP)])
    pltpu.sync_copy(dstp_hbm.at[pl.ds(wid * EP, EP)], didx.at[pl.ds(0, EP)])
    pltpu.sync_copy(dstn_hbm.at[pl.ds(wid * EP, EP)], didx.at[pl.ds(EP, EP)])

    def start_gathers(c, b):
        for j in range(NSUBC):
            off = c * CHE + j * SUBK
            pltpu.async_copy(z_hbm.at[sidx.at[pl.ds(off, SUBK)]],
                             zs.at[b].at[pl.ds(j * SUBK, SUBK)], gsem.at[b])
            pltpu.async_copy(z_hbm.at[didx.at[pl.ds(off, SUBK)]],
                             zd.at[b].at[pl.ds(j * SUBK, SUBK)], gsem.at[b])

    start_gathers(0, 0)
    start_gathers(1, 1)

    def body(c, carry):
        b = lax.rem(c, 2)
        pltpu.make_async_copy(z_hbm.at[pl.ds(0, CHE)], zs.at[b],
                              gsem.at[b]).wait()
        pltpu.make_async_copy(z_hbm.at[pl.ds(0, CHE)], zd.at[b],
                              gsem.at[b]).wait()

        def gb(g, cc):
            row0 = g * L
            for k in range(L):
                a1 = zs[b, row0 + k, pl.ds(0, L)]
                a2 = zs[b, row0 + k, pl.ds(L, L)]
                d1 = zd[b, row0 + k, pl.ds(0, L)]
                d2 = zd[b, row0 + k, pl.ds(L, L)]
                ptmp[k, pl.ds(0, L)] = a1 * d1 + a2 * d2
            q = zeros16
            for col in range(L):
                colv = jnp.full((L,), col, jnp.int32)
                q = q + plsc.load_gather(ptmp, [iota, colv])
            lbuf[pl.ds(c * CHE + g * L, L)] = q
            return cc
        lax.fori_loop(0, CHE // L, gb, 0)

        @pl.when(c + 2 < NCH_D)
        def _():
            start_gathers(c + 2, b)
        return carry
    lax.fori_loop(0, NCH_D, body, 0)

    pltpu.sync_copy(lbuf.at[pl.ds(0, EP)], out_hbm.at[0, wid])
    pltpu.sync_copy(lbuf.at[pl.ds(EP, EP)], out_hbm.at[1, wid])


_dec_call = pl.kernel(
    _sc_dec_body,
    out_type=jax.ShapeDtypeStruct((2, NW, EP), jnp.float32),
    mesh=_MESH,
    compiler_params=_SC_PARAMS,
    scratch_types=[
        pltpu.VMEM((EPD,), jnp.int32),
        pltpu.VMEM((EPD,), jnp.int32),
        pltpu.VMEM((2, CHE, ZD), jnp.float32),
        pltpu.VMEM((2, CHE, ZD), jnp.float32),
        pltpu.VMEM((EPD,), jnp.float32),
        pltpu.VMEM((L, PTP), jnp.float32),
        pltpu.SemaphoreType.DMA((2,)),
    ],
)


def _tc_prep_body(counts_ref, x_ref, w1_ref, hs_ref, dinv_ref):
    deg = jnp.sum(counts_ref[0], axis=0) + 1.0
    dinv = lax.rsqrt(deg)
    h = jnp.dot(x_ref[...], w1_ref[...], preferred_element_type=jnp.float32)
    hs_ref[...] = h * dinv[:, None]
    dinv_ref[...] = dinv[:, None]


def _tc_h_body(t_ref, hs_ref, dinv_ref, b1_ref, out_ref):
    t = t_ref[0] + t_ref[1] + hs_ref[...]
    dinv = dinv_ref[...]
    h = jnp.maximum(t * dinv + b1_ref[...], 0.0)
    out_ref[...] = h * dinv


def _tc_z_body(t_ref, hs2_ref, dinv_ref, eps_ref, wmu_ref, bmu_ref, wlv_ref,
               blv_ref, z_ref, kl_ref):
    i = pl.program_id(0)
    agg = (t_ref[0] + t_ref[1] + hs2_ref[...]) * dinv_ref[...]
    mu = jnp.dot(agg, wmu_ref[...], preferred_element_type=jnp.float32) + bmu_ref[...]
    lv = jnp.dot(agg, wlv_ref[...], preferred_element_type=jnp.float32) + blv_ref[...]
    z_ref[...] = mu + eps_ref[...] * jnp.exp(0.5 * lv)
    klp = jnp.sum(1.0 + lv - mu * mu - jnp.exp(lv)).reshape(1, 1)

    @pl.when(i == 0)
    def _():
        kl_ref[...] = klp

    @pl.when(i > 0)
    def _():
        kl_ref[...] = kl_ref[...] + klp


def _tc_loss_body(lg_ref, kl_ref, loss_ref, recon_ref, klo_ref):
    i = pl.program_id(0)
    l = lg_ref[0]
    lab = (i == 0).astype(jnp.float32)
    s = jnp.sum(jnp.maximum(l, 0.0) - l * lab
                + jnp.log1p(jnp.exp(-jnp.abs(l)))).reshape(1, 1)

    @pl.when(i == 0)
    def _():
        loss_ref[...] = s

    @pl.when(i == 1)
    def _():
        recon = (loss_ref[...] + s) / (2.0 * E)
        kl = -0.5 * kl_ref[...] / (N * ZD)
        loss_ref[...] = recon + kl
        recon_ref[...] = recon
        klo_ref[...] = kl


def kernel(x, edge_index, neg_edge_index, eps, W1, b1, Wmu, bmu, Wlv, blv):
    src = edge_index[0].astype(jnp.int32)
    dst = edge_index[1].astype(jnp.int32)

    counts = _deg_call(dst)

    hs1, dinv = pl.pallas_call(
        _tc_prep_body,
        grid=(GRID,),
        in_specs=[
            pl.BlockSpec((1, NW, RB), lambda i: (i, 0, 0)),
            pl.BlockSpec((RB, IN_DIM), lambda i: (i, 0)),
            pl.BlockSpec((IN_DIM, HID), lambda i: (0, 0)),
        ],
        out_specs=[
            pl.BlockSpec((RB, HID), lambda i: (i, 0)),
            pl.BlockSpec((RB, 1), lambda i: (i, 0)),
        ],
        out_shape=[
            jax.ShapeDtypeStruct((N, HID), jnp.float32),
            jax.ShapeDtypeStruct((N, 1), jnp.float32),
        ],
    )(counts, x, W1)

    t1 = _agg_call(hs1, src, dst)

    hs2 = pl.pallas_call(
        _tc_h_body,
        grid=(GRID,),
        in_specs=[
            pl.BlockSpec((NC, RB, HID), lambda i: (0, i, 0)),
            pl.BlockSpec((RB, HID), lambda i: (i, 0)),
            pl.BlockSpec((RB, 1), lambda i: (i, 0)),
            pl.BlockSpec((1, HID), lambda i: (0, 0)),
        ],
        out_specs=pl.BlockSpec((RB, HID), lambda i: (i, 0)),
        out_shape=jax.ShapeDtypeStruct((N, HID), jnp.float32),
    )(t1, hs1, dinv, b1.reshape(1, HID))

    t2 = _agg_call(hs2, src, dst)

    z, klsum = pl.pallas_call(
        _tc_z_body,
        grid=(GRID,),
        in_specs=[
            pl.BlockSpec((NC, RB, HID), lambda i: (0, i, 0)),
            pl.BlockSpec((RB, HID), lambda i: (i, 0)),
            pl.BlockSpec((RB, 1), lambda i: (i, 0)),
            pl.BlockSpec((RB, ZD), lambda i: (i, 0)),
            pl.BlockSpec((HID, ZD), lambda i: (0, 0)),
            pl.BlockSpec((1, ZD), lambda i: (0, 0)),
            pl.BlockSpec((HID, ZD), lambda i: (0, 0)),
            pl.BlockSpec((1, ZD), lambda i: (0, 0)),
        ],
        out_specs=[
            pl.BlockSpec((RB, ZD), lambda i: (i, 0)),
            pl.BlockSpec((1, 1), lambda i: (0, 0)),
        ],
        out_shape=[
            jax.ShapeDtypeStruct((N, ZD), jnp.float32),
            jax.ShapeDtypeStruct((1, 1), jnp.float32),
        ],
    )(t2, hs2, dinv, eps, Wmu, bmu.reshape(1, ZD), Wlv, blv.reshape(1, ZD))

    logits2 = _dec_call(z, src, dst, neg_edge_index[0].astype(jnp.int32),
                        neg_edge_index[1].astype(jnp.int32))

    loss, recon, kl = pl.pallas_call(
        _tc_loss_body,
        grid=(2,),
        in_specs=[
            pl.BlockSpec((1, NW, EP), lambda i: (i, 0, 0)),
            pl.BlockSpec((1, 1), lambda i: (0, 0)),
        ],
        out_specs=[
            pl.BlockSpec((1, 1), lambda i: (0, 0)),
            pl.BlockSpec((1, 1), lambda i: (0, 0)),
            pl.BlockSpec((1, 1), lambda i: (0, 0)),
        ],
        out_shape=[
            jax.ShapeDtypeStruct((1, 1), jnp.float32),
            jax.ShapeDtypeStruct((1, 1), jnp.float32),
            jax.ShapeDtypeStruct((1, 1), jnp.float32),
        ],
    )(logits2, klsum)

    return (loss.reshape(()),
            jax.lax.stop_gradient(recon.reshape(())),
            jax.lax.stop_gradient(kl.reshape(())))


# speedup vs baseline: 1.0472x; 1.0472x over previous
import jax
import jax.numpy as jnp
from jax import lax
from jax.experimental import pallas as pl
from jax.experimental.pallas import tpu as pltpu
from jax.experimental.pallas import tpu_sc as plsc

N = 10000
E = 320000
IN_DIM = 128
HID = 64
ZD = 32

NC, NS, L = 2, 16, 16
NW = NC * NS
EP = E // NW
EPD = 2 * E // NW
SUBK = 80
NSUBC = 5
CHE = SUBK * NSUBC
NCH_A = EP // CHE
NCH_D = EPD // CHE
NPAD = 10240
NPT = NPAD // NS
ZROWS = 160

RB = 1000
GRID = N // RB

_MESH = plsc.VectorSubcoreMesh(
    core_axis_name="c", subcore_axis_name="s", num_cores=NC, num_subcores=NS)
_SC_PARAMS = pltpu.CompilerParams(needs_layout_passes=False,
                                  use_tc_tiling_on_sc=False)


def _sc_deg_body(dst_hbm, out_hbm, dstbuf, countbuf):
    cid = lax.axis_index("c")
    sid = lax.axis_index("s")
    wid = sid * NC + cid
    zeros16 = jnp.zeros((L,), jnp.float32)
    ones16 = jnp.ones((L,), jnp.float32)

    def zb(i, c):
        countbuf[pl.ds(i * L, L)] = zeros16
        return c
    lax.fori_loop(0, N // L, zb, 0)

    pltpu.sync_copy(dst_hbm.at[pl.ds(wid * EP, EP)], dstbuf)

    def cb(i, c):
        idx = dstbuf[pl.ds(i * L, L)]
        plsc.addupdate_scatter(countbuf, [idx], ones16)
        return c
    lax.fori_loop(0, EP // L, cb, 0)

    for g in range(GRID):
        pltpu.sync_copy(countbuf.at[pl.ds(g * RB, RB)], out_hbm.at[g, wid])


_deg_call = pl.kernel(
    _sc_deg_body,
    out_type=jax.ShapeDtypeStruct((GRID, NW, RB), jnp.float32),
    mesh=_MESH,
    compiler_params=_SC_PARAMS,
    scratch_types=[
        pltpu.VMEM((EP,), jnp.int32),
        pltpu.VMEM((N,), jnp.float32),
    ],
)


def _sc_agg_body(tab_hbm, src_hbm, dst_hbm, out_hbm, sidx, didx, rows, zbuf,
                 acc, gsem, ssem):
    cid = lax.axis_index("c")
    sid = lax.axis_index("s")
    wid = sid * NC + cid
    zeros16 = jnp.zeros((L,), jnp.float32)

    def zb(i, c):
        for j in range(HID // L):
            zbuf[i, pl.ds(j * L, L)] = zeros16
        return c
    lax.fori_loop(0, ZROWS, zb, 0)
    for r in range(NPT // ZROWS):
        pltpu.sync_copy(zbuf, acc.at[pl.ds(sid * NPT + r * ZROWS, ZROWS)])
    plsc.subcore_barrier()

    pltpu.sync_copy(src_hbm.at[pl.ds(wid * EP, EP)], sidx)
    pltpu.sync_copy(dst_hbm.at[pl.ds(wid * EP, EP)], didx)

    def start_gathers(c, b):
        for j in range(NSUBC):
            pltpu.async_copy(
                tab_hbm.at[sidx.at[pl.ds(c * CHE + j * SUBK, SUBK)]],
                rows.at[b].at[pl.ds(j * SUBK, SUBK)], gsem.at[b])

    start_gathers(0, 0)
    start_gathers(1, 1)

    def body(c, carry):
        b = lax.rem(c, 2)
        pltpu.make_async_copy(tab_hbm.at[pl.ds(0, CHE)], rows.at[b],
                              gsem.at[b]).wait()
        for j in range(NSUBC):
            pltpu.async_copy(rows.at[b].at[pl.ds(j * SUBK, SUBK)],
                             acc.at[didx.at[pl.ds((c * NSUBC + j) * SUBK, SUBK)]],
                             ssem.at[b], add=True)
        pltpu.make_async_copy(tab_hbm.at[pl.ds(0, CHE)], rows.at[b],
                              ssem.at[b]).wait()

        @pl.when(c + 2 < NCH_A)
        def _():
            start_gathers(c + 2, b)
        return carry
    lax.fori_loop(0, NCH_A, body, 0)

    plsc.subcore_barrier()
    pltpu.sync_copy(acc.at[pl.ds(sid * NPT, NPT)],
                    out_hbm.at[cid, pl.ds(sid * NPT, NPT)])


_agg_call = pl.kernel(
    _sc_agg_body,
    out_type=jax.ShapeDtypeStruct((NC, NPAD, HID), jnp.float32),
    mesh=_MESH,
    compiler_params=_SC_PARAMS,
    scratch_types=[
        pltpu.VMEM((EP,), jnp.int32),
        pltpu.VMEM((EP,), jnp.int32),
        pltpu.VMEM((2, CHE, HID), jnp.float32),
        pltpu.VMEM((ZROWS, HID), jnp.float32),
        pltpu.VMEM_SHARED((NPAD, HID), jnp.float32),
        pltpu.SemaphoreType.DMA((2,)),
        pltpu.SemaphoreType.DMA((2,)),
    ],
)


PTP = 24


def _sc_dec_body(z_hbm, srcp_hbm, dstp_hbm, srcn_hbm, dstn_hbm, out_hbm,
                 sidx, didx, zs, zd, lbuf, ptmp, gsem):
    cid = lax.axis_index("c")
    sid = lax.axis_index("s")
    wid = sid * NC + cid
    iota = lax.iota(jnp.int32, L)
    zeros16 = jnp.zeros((L,), jnp.float32)

    pltpu.sync_copy(srcp_hbm.at[pl.ds(wid * EP, EP)], sidx.at[pl.ds(0, EP)])
    pltpu.sync_copy(srcn_hbm.at[pl.ds(wid * EP, EP)], sidx.at[pl.ds(EP, EP)])
    pltpu.sync_copy(dstp_hbm.at[pl.ds(wid * EP, EP)], didx.at[pl.ds(0, EP)])
    pltpu.sync_copy(dstn_hbm.at[pl.ds(wid * EP, EP)], didx.at[pl.ds(EP, EP)])

    def start_gathers(c, b):
        for j in range(NSUBC):
            off = c * CHE + j * SUBK
            pltpu.async_copy(z_hbm.at[sidx.at[pl.ds(off, SUBK)]],
                             zs.at[b].at[pl.ds(j * SUBK, SUBK)], gsem.at[b])
            pltpu.async_copy(z_hbm.at[didx.at[pl.ds(off, SUBK)]],
                             zd.at[b].at[pl.ds(j * SUBK, SUBK)], gsem.at[b])

    start_gathers(0, 0)
    start_gathers(1, 1)

    def body(c, carry):
        b = lax.rem(c, 2)
        pltpu.make_async_copy(z_hbm.at[pl.ds(0, CHE)], zs.at[b],
                              gsem.at[b]).wait()
        pltpu.make_async_copy(z_hbm.at[pl.ds(0, CHE)], zd.at[b],
                              gsem.at[b]).wait()

        def gb(g, cc):
            row0 = g * L
            for k in range(L):
                a = zs[b, row0 + k, :]
                d = zd[b, row0 + k, :]
                pe, po = plsc.unpack(a * d, format=plsc.PackFormat.INTERLEAVED)
                ptmp[k, pl.ds(0, L)] = pe + po
            qs = []
            for col in range(L):
                colv = jnp.full((L,), col, jnp.int32)
                qs.append(plsc.load_gather(ptmp, [iota, colv]))
            while len(qs) > 1:
                qs = [qs[i] + qs[i + 1] for i in range(0, len(qs) - 1, 2)]                      + ([qs[-1]] if len(qs) % 2 else [])
            lbuf[pl.ds(c * CHE + g * L, L)] = qs[0]
            return cc
        lax.fori_loop(0, CHE // L, gb, 0)

        @pl.when(c + 2 < NCH_D)
        def _():
            start_gathers(c + 2, b)
        return carry
    lax.fori_loop(0, NCH_D, body, 0)

    pltpu.sync_copy(lbuf.at[pl.ds(0, EP)], out_hbm.at[0, wid])
    pltpu.sync_copy(lbuf.at[pl.ds(EP, EP)], out_hbm.at[1, wid])


_dec_call = pl.kernel(
    _sc_dec_body,
    out_type=jax.ShapeDtypeStruct((2, NW, EP), jnp.float32),
    mesh=_MESH,
    compiler_params=_SC_PARAMS,
    scratch_types=[
        pltpu.VMEM((EPD,), jnp.int32),
        pltpu.VMEM((EPD,), jnp.int32),
        pltpu.VMEM((2, CHE, ZD), jnp.bfloat16),
        pltpu.VMEM((2, CHE, ZD), jnp.bfloat16),
        pltpu.VMEM((EPD,), jnp.float32),
        pltpu.VMEM((L, PTP), jnp.float32),
        pltpu.SemaphoreType.DMA((2,)),
    ],
)


def _tc_prep_body(counts_ref, x_ref, w1_ref, hs_ref, dinv_ref):
    deg = jnp.sum(counts_ref[0], axis=0) + 1.0
    dinv = lax.rsqrt(deg)
    h = jnp.dot(x_ref[...], w1_ref[...], preferred_element_type=jnp.float32)
    hs_ref[...] = h * dinv[:, None]
    dinv_ref[...] = dinv[:, None]


def _tc_h_body(t_ref, hs_ref, dinv_ref, b1_ref, out_ref):
    t = t_ref[0] + t_ref[1] + hs_ref[...]
    dinv = dinv_ref[...]
    h = jnp.maximum(t * dinv + b1_ref[...], 0.0)
    out_ref[...] = h * dinv


def _tc_z_body(t_ref, hs2_ref, dinv_ref, eps_ref, wmu_ref, bmu_ref, wlv_ref,
               blv_ref, z_ref, kl_ref):
    i = pl.program_id(0)
    agg = (t_ref[0] + t_ref[1] + hs2_ref[...]) * dinv_ref[...]
    mu = jnp.dot(agg, wmu_ref[...], preferred_element_type=jnp.float32) + bmu_ref[...]
    lv = jnp.dot(agg, wlv_ref[...], preferred_element_type=jnp.float32) + blv_ref[...]
    z_ref[...] = (mu + eps_ref[...] * jnp.exp(0.5 * lv)).astype(jnp.bfloat16)
    klp = jnp.sum(1.0 + lv - mu * mu - jnp.exp(lv)).reshape(1, 1)

    @pl.when(i == 0)
    def _():
        kl_ref[...] = klp

    @pl.when(i > 0)
    def _():
        kl_ref[...] = kl_ref[...] + klp


def _tc_loss_body(lg_ref, kl_ref, loss_ref, recon_ref, klo_ref):
    i = pl.program_id(0)
    l = lg_ref[0]
    lab = (i == 0).astype(jnp.float32)
    s = jnp.sum(jnp.maximum(l, 0.0) - l * lab
                + jnp.log1p(jnp.exp(-jnp.abs(l)))).reshape(1, 1)

    @pl.when(i == 0)
    def _():
        loss_ref[...] = s

    @pl.when(i == 1)
    def _():
        recon = (loss_ref[...] + s) / (2.0 * E)
        kl = -0.5 * kl_ref[...] / (N * ZD)
        loss_ref[...] = recon + kl
        recon_ref[...] = recon
        klo_ref[...] = kl


def kernel(x, edge_index, neg_edge_index, eps, W1, b1, Wmu, bmu, Wlv, blv):
    src = edge_index[0].astype(jnp.int32)
    dst = edge_index[1].astype(jnp.int32)

    counts = _deg_call(dst)

    hs1, dinv = pl.pallas_call(
        _tc_prep_body,
        grid=(GRID,),
        in_specs=[
            pl.BlockSpec((1, NW, RB), lambda i: (i, 0, 0)),
            pl.BlockSpec((RB, IN_DIM), lambda i: (i, 0)),
            pl.BlockSpec((IN_DIM, HID), lambda i: (0, 0)),
        ],
        out_specs=[
            pl.BlockSpec((RB, HID), lambda i: (i, 0)),
            pl.BlockSpec((RB, 1), lambda i: (i, 0)),
        ],
        out_shape=[
            jax.ShapeDtypeStruct((N, HID), jnp.float32),
            jax.ShapeDtypeStruct((N, 1), jnp.float32),
        ],
    )(counts, x, W1)

    t1 = _agg_call(hs1, src, dst)

    hs2 = pl.pallas_call(
        _tc_h_body,
        grid=(GRID,),
        in_specs=[
            pl.BlockSpec((NC, RB, HID), lambda i: (0, i, 0)),
            pl.BlockSpec((RB, HID), lambda i: (i, 0)),
            pl.BlockSpec((RB, 1), lambda i: (i, 0)),
            pl.BlockSpec((1, HID), lambda i: (0, 0)),
        ],
        out_specs=pl.BlockSpec((RB, HID), lambda i: (i, 0)),
        out_shape=jax.ShapeDtypeStruct((N, HID), jnp.float32),
    )(t1, hs1, dinv, b1.reshape(1, HID))

    t2 = _agg_call(hs2, src, dst)

    z, klsum = pl.pallas_call(
        _tc_z_body,
        grid=(GRID,),
        in_specs=[
            pl.BlockSpec((NC, RB, HID), lambda i: (0, i, 0)),
            pl.BlockSpec((RB, HID), lambda i: (i, 0)),
            pl.BlockSpec((RB, 1), lambda i: (i, 0)),
            pl.BlockSpec((RB, ZD), lambda i: (i, 0)),
            pl.BlockSpec((HID, ZD), lambda i: (0, 0)),
            pl.BlockSpec((1, ZD), lambda i: (0, 0)),
            pl.BlockSpec((HID, ZD), lambda i: (0, 0)),
            pl.BlockSpec((1, ZD), lambda i: (0, 0)),
        ],
        out_specs=[
            pl.BlockSpec((RB, ZD), lambda i: (i, 0)),
            pl.BlockSpec((1, 1), lambda i: (0, 0)),
        ],
        out_shape=[
            jax.ShapeDtypeStruct((N, ZD), jnp.bfloat16),
            jax.ShapeDtypeStruct((1, 1), jnp.float32),
        ],
    )(t2, hs2, dinv, eps, Wmu, bmu.reshape(1, ZD), Wlv, blv.reshape(1, ZD))

    logits2 = _dec_call(z, src, dst, neg_edge_index[0].astype(jnp.int32),
                        neg_edge_index[1].astype(jnp.int32))

    loss, recon, kl = pl.pallas_call(
        _tc_loss_body,
        grid=(2,),
        in_specs=[
            pl.BlockSpec((1, NW, EP), lambda i: (i, 0, 0)),
            pl.BlockSpec((1, 1), lambda i: (0, 0)),
        ],
        out_specs=[
            pl.BlockSpec((1, 1), lambda i: (0, 0)),
            pl.BlockSpec((1, 1), lambda i: (0, 0)),
            pl.BlockSpec((1, 1), lambda i: (0, 0)),
        ],
        out_shape=[
            jax.ShapeDtypeStruct((1, 1), jnp.float32),
            jax.ShapeDtypeStruct((1, 1), jnp.float32),
            jax.ShapeDtypeStruct((1, 1), jnp.float32),
        ],
    )(logits2, klsum)

    return (loss.reshape(()),
            jax.lax.stop_gradient(recon.reshape(())),
            jax.lax.stop_gradient(kl.reshape(())))


# speedup vs baseline: 1.0823x; 1.0335x over previous
import jax
import jax.numpy as jnp
from jax import lax
from jax.experimental import pallas as pl
from jax.experimental.pallas import tpu as pltpu
from jax.experimental.pallas import tpu_sc as plsc

N = 10000
E = 320000
IN_DIM = 128
HID = 64
ZD = 32

NC, NS, L = 2, 16, 16
NW = NC * NS
EP = E // NW
EPD = 2 * E // NW
SUBK = 80
NSUBC = 5
CHE = SUBK * NSUBC
NCH_A = EP // CHE
NCH_D = EPD // CHE
NPAD = 10240
NPT = NPAD // NS
ZROWS = 160

RB = 1000
GRID = N // RB

_MESH = plsc.VectorSubcoreMesh(
    core_axis_name="c", subcore_axis_name="s", num_cores=NC, num_subcores=NS)
_SC_PARAMS = pltpu.CompilerParams(needs_layout_passes=False,
                                  use_tc_tiling_on_sc=False)


def _sc_deg_body(dst_hbm, out_hbm, dstbuf, countbuf):
    cid = lax.axis_index("c")
    sid = lax.axis_index("s")
    wid = sid * NC + cid
    zeros16 = jnp.zeros((L,), jnp.float32)
    ones16 = jnp.ones((L,), jnp.float32)

    def zb(i, c):
        countbuf[pl.ds(i * L, L)] = zeros16
        return c
    lax.fori_loop(0, N // L, zb, 0)

    pltpu.sync_copy(dst_hbm.at[pl.ds(wid * EP, EP)], dstbuf)

    def cb(i, c):
        idx = dstbuf[pl.ds(i * L, L)]
        plsc.addupdate_scatter(countbuf, [idx], ones16)
        return c
    lax.fori_loop(0, EP // L, cb, 0)

    for g in range(GRID):
        pltpu.sync_copy(countbuf.at[pl.ds(g * RB, RB)], out_hbm.at[g, wid])


_deg_call = pl.kernel(
    _sc_deg_body,
    out_type=jax.ShapeDtypeStruct((GRID, NW, RB), jnp.float32),
    mesh=_MESH,
    compiler_params=_SC_PARAMS,
    scratch_types=[
        pltpu.VMEM((EP,), jnp.int32),
        pltpu.VMEM((N,), jnp.float32),
    ],
)


def _sc_agg_body(tab_hbm, src_hbm, dst_hbm, out_hbm, sidx, didx, rows, zbuf,
                 acc, gsem, ssem):
    cid = lax.axis_index("c")
    sid = lax.axis_index("s")
    wid = sid * NC + cid
    zeros16 = jnp.zeros((L,), jnp.float32)

    def zb(i, c):
        for j in range(HID // L):
            zbuf[i, pl.ds(j * L, L)] = zeros16
        return c
    lax.fori_loop(0, ZROWS, zb, 0)
    for r in range(NPT // ZROWS):
        pltpu.sync_copy(zbuf, acc.at[pl.ds(sid * NPT + r * ZROWS, ZROWS)])
    plsc.subcore_barrier()

    pltpu.sync_copy(src_hbm.at[pl.ds(wid * EP, EP)], sidx)
    pltpu.sync_copy(dst_hbm.at[pl.ds(wid * EP, EP)], didx)

    def start_gathers(c, b):
        for j in range(NSUBC):
            pltpu.async_copy(
                tab_hbm.at[sidx.at[pl.ds(c * CHE + j * SUBK, SUBK)]],
                rows.at[b].at[pl.ds(j * SUBK, SUBK)], gsem.at[b])

    start_gathers(0, 0)
    start_gathers(1, 1)

    def body(c, carry):
        b = lax.rem(c, 2)
        pltpu.make_async_copy(tab_hbm.at[pl.ds(0, CHE)], rows.at[b],
                              gsem.at[b]).wait()
        for j in range(NSUBC):
            pltpu.async_copy(rows.at[b].at[pl.ds(j * SUBK, SUBK)],
                             acc.at[didx.at[pl.ds((c * NSUBC + j) * SUBK, SUBK)]],
                             ssem.at[b], add=True)
        pltpu.make_async_copy(tab_hbm.at[pl.ds(0, CHE)], rows.at[b],
                              ssem.at[b]).wait()

        @pl.when(c + 2 < NCH_A)
        def _():
            start_gathers(c + 2, b)
        return carry
    lax.fori_loop(0, NCH_A, body, 0)

    plsc.subcore_barrier()
    pltpu.sync_copy(acc.at[pl.ds(sid * NPT, NPT)],
                    out_hbm.at[cid, pl.ds(sid * NPT, NPT)])


_agg_call = pl.kernel(
    _sc_agg_body,
    out_type=jax.ShapeDtypeStruct((NC, NPAD, HID), jnp.float32),
    mesh=_MESH,
    compiler_params=_SC_PARAMS,
    scratch_types=[
        pltpu.VMEM((EP,), jnp.int32),
        pltpu.VMEM((EP,), jnp.int32),
        pltpu.VMEM((2, CHE, HID), jnp.float32),
        pltpu.VMEM((ZROWS, HID), jnp.float32),
        pltpu.VMEM_SHARED((NPAD, HID), jnp.float32),
        pltpu.SemaphoreType.DMA((2,)),
        pltpu.SemaphoreType.DMA((2,)),
    ],
)


PTP = 24


def _sc_dec_body(z_hbm, srcp_hbm, dstp_hbm, srcn_hbm, dstn_hbm, out_hbm,
                 sidx, didx, zs, zd, lbuf, ptmp, gsem):
    cid = lax.axis_index("c")
    sid = lax.axis_index("s")
    wid = sid * NC + cid
    iota = lax.iota(jnp.int32, L)
    zeros16 = jnp.zeros((L,), jnp.float32)

    pltpu.sync_copy(srcp_hbm.at[pl.ds(wid * EP, EP)], sidx.at[pl.ds(0, EP)])
    pltpu.sync_copy(srcn_hbm.at[pl.ds(wid * EP, EP)], sidx.at[pl.ds(EP, EP)])
    pltpu.sync_copy(dstp_hbm.at[pl.ds(wid * EP, EP)], didx.at[pl.ds(0, EP)])
    pltpu.sync_copy(dstn_hbm.at[pl.ds(wid * EP, EP)], didx.at[pl.ds(EP, EP)])

    def start_gathers(c, b):
        for j in range(NSUBC):
            off = c * CHE + j * SUBK
            pltpu.async_copy(z_hbm.at[sidx.at[pl.ds(off, SUBK)]],
                             zs.at[b].at[pl.ds(j * SUBK, SUBK)], gsem.at[b])
            pltpu.async_copy(z_hbm.at[didx.at[pl.ds(off, SUBK)]],
                             zd.at[b].at[pl.ds(j * SUBK, SUBK)], gsem.at[b])

    start_gathers(0, 0)
    start_gathers(1, 1)

    def body(c, carry):
        b = lax.rem(c, 2)
        pltpu.make_async_copy(z_hbm.at[pl.ds(0, CHE)], zs.at[b],
                              gsem.at[b]).wait()
        pltpu.make_async_copy(z_hbm.at[pl.ds(0, CHE)], zd.at[b],
                              gsem.at[b]).wait()

        def gb(g5, cc):
            for sl in range(5):
                g = g5 * 5 + sl
                row0 = g * L
                for k in range(L):
                    a = zs[b, row0 + k, :]
                    d = zd[b, row0 + k, :]
                    pe, po = plsc.unpack(a * d,
                                         format=plsc.PackFormat.INTERLEAVED)
                    ptmp[sl, k, pl.ds(0, L)] = pe + po
                qs = []
                for col in range(L):
                    colv = jnp.full((L,), col, jnp.int32)
                    qs.append(plsc.load_gather(ptmp.at[sl], [iota, colv]))
                while len(qs) > 1:
                    qs = [qs[i] + qs[i + 1] for i in range(0, len(qs) - 1, 2)]                          + ([qs[-1]] if len(qs) % 2 else [])
                lbuf[pl.ds(c * CHE + g * L, L)] = qs[0]
            return cc
        lax.fori_loop(0, (CHE // L) // 5, gb, 0)

        @pl.when(c + 2 < NCH_D)
        def _():
            start_gathers(c + 2, b)
        return carry
    lax.fori_loop(0, NCH_D, body, 0)

    pltpu.sync_copy(lbuf.at[pl.ds(0, EP)], out_hbm.at[0, wid])
    pltpu.sync_copy(lbuf.at[pl.ds(EP, EP)], out_hbm.at[1, wid])


_dec_call = pl.kernel(
    _sc_dec_body,
    out_type=jax.ShapeDtypeStruct((2, NW, EP), jnp.float32),
    mesh=_MESH,
    compiler_params=_SC_PARAMS,
    scratch_types=[
        pltpu.VMEM((EPD,), jnp.int32),
        pltpu.VMEM((EPD,), jnp.int32),
        pltpu.VMEM((2, CHE, ZD), jnp.bfloat16),
        pltpu.VMEM((2, CHE, ZD), jnp.bfloat16),
        pltpu.VMEM((EPD,), jnp.float32),
        pltpu.VMEM((5, L, PTP), jnp.float32),
        pltpu.SemaphoreType.DMA((2,)),
    ],
)


def _tc_prep_body(counts_ref, x_ref, w1_ref, hs_ref, dinv_ref):
    deg = jnp.sum(counts_ref[0], axis=0) + 1.0
    dinv = lax.rsqrt(deg)
    h = jnp.dot(x_ref[...], w1_ref[...], preferred_element_type=jnp.float32)
    hs_ref[...] = h * dinv[:, None]
    dinv_ref[...] = dinv[:, None]


def _tc_h_body(t_ref, hs_ref, dinv_ref, b1_ref, out_ref):
    t = t_ref[0] + t_ref[1] + hs_ref[...]
    dinv = dinv_ref[...]
    h = jnp.maximum(t * dinv + b1_ref[...], 0.0)
    out_ref[...] = h * dinv


def _tc_z_body(t_ref, hs2_ref, dinv_ref, eps_ref, wmu_ref, bmu_ref, wlv_ref,
               blv_ref, z_ref, kl_ref):
    i = pl.program_id(0)
    agg = (t_ref[0] + t_ref[1] + hs2_ref[...]) * dinv_ref[...]
    mu = jnp.dot(agg, wmu_ref[...], preferred_element_type=jnp.float32) + bmu_ref[...]
    lv = jnp.dot(agg, wlv_ref[...], preferred_element_type=jnp.float32) + blv_ref[...]
    z_ref[...] = (mu + eps_ref[...] * jnp.exp(0.5 * lv)).astype(jnp.bfloat16)
    klp = jnp.sum(1.0 + lv - mu * mu - jnp.exp(lv)).reshape(1, 1)

    @pl.when(i == 0)
    def _():
        kl_ref[...] = klp

    @pl.when(i > 0)
    def _():
        kl_ref[...] = kl_ref[...] + klp


def _tc_loss_body(lg_ref, kl_ref, loss_ref, recon_ref, klo_ref):
    i = pl.program_id(0)
    l = lg_ref[0]
    lab = (i == 0).astype(jnp.float32)
    s = jnp.sum(jnp.maximum(l, 0.0) - l * lab
                + jnp.log1p(jnp.exp(-jnp.abs(l)))).reshape(1, 1)

    @pl.when(i == 0)
    def _():
        loss_ref[...] = s

    @pl.when(i == 1)
    def _():
        recon = (loss_ref[...] + s) / (2.0 * E)
        kl = -0.5 * kl_ref[...] / (N * ZD)
        loss_ref[...] = recon + kl
        recon_ref[...] = recon
        klo_ref[...] = kl


def kernel(x, edge_index, neg_edge_index, eps, W1, b1, Wmu, bmu, Wlv, blv):
    src = edge_index[0].astype(jnp.int32)
    dst = edge_index[1].astype(jnp.int32)

    counts = _deg_call(dst)

    hs1, dinv = pl.pallas_call(
        _tc_prep_body,
        grid=(GRID,),
        in_specs=[
            pl.BlockSpec((1, NW, RB), lambda i: (i, 0, 0)),
            pl.BlockSpec((RB, IN_DIM), lambda i: (i, 0)),
            pl.BlockSpec((IN_DIM, HID), lambda i: (0, 0)),
        ],
        out_specs=[
            pl.BlockSpec((RB, HID), lambda i: (i, 0)),
            pl.BlockSpec((RB, 1), lambda i: (i, 0)),
        ],
        out_shape=[
            jax.ShapeDtypeStruct((N, HID), jnp.float32),
            jax.ShapeDtypeStruct((N, 1), jnp.float32),
        ],
    )(counts, x, W1)

    t1 = _agg_call(hs1, src, dst)

    hs2 = pl.pallas_call(
        _tc_h_body,
        grid=(GRID,),
        in_specs=[
            pl.BlockSpec((NC, RB, HID), lambda i: (0, i, 0)),
            pl.BlockSpec((RB, HID), lambda i: (i, 0)),
            pl.BlockSpec((RB, 1), lambda i: (i, 0)),
            pl.BlockSpec((1, HID), lambda i: (0, 0)),
        ],
        out_specs=pl.BlockSpec((RB, HID), lambda i: (i, 0)),
        out_shape=jax.ShapeDtypeStruct((N, HID), jnp.float32),
    )(t1, hs1, dinv, b1.reshape(1, HID))

    t2 = _agg_call(hs2, src, dst)

    z, klsum = pl.pallas_call(
        _tc_z_body,
        grid=(GRID,),
        in_specs=[
            pl.BlockSpec((NC, RB, HID), lambda i: (0, i, 0)),
            pl.BlockSpec((RB, HID), lambda i: (i, 0)),
            pl.BlockSpec((RB, 1), lambda i: (i, 0)),
            pl.BlockSpec((RB, ZD), lambda i: (i, 0)),
            pl.BlockSpec((HID, ZD), lambda i: (0, 0)),
            pl.BlockSpec((1, ZD), lambda i: (0, 0)),
            pl.BlockSpec((HID, ZD), lambda i: (0, 0)),
            pl.BlockSpec((1, ZD), lambda i: (0, 0)),
        ],
        out_specs=[
            pl.BlockSpec((RB, ZD), lambda i: (i, 0)),
            pl.BlockSpec((1, 1), lambda i: (0, 0)),
        ],
        out_shape=[
            jax.ShapeDtypeStruct((N, ZD), jnp.bfloat16),
            jax.ShapeDtypeStruct((1, 1), jnp.float32),
        ],
    )(t2, hs2, dinv, eps, Wmu, bmu.reshape(1, ZD), Wlv, blv.reshape(1, ZD))

    logits2 = _dec_call(z, src, dst, neg_edge_index[0].astype(jnp.int32),
                        neg_edge_index[1].astype(jnp.int32))

    loss, recon, kl = pl.pallas_call(
        _tc_loss_body,
        grid=(2,),
        in_specs=[
            pl.BlockSpec((1, NW, EP), lambda i: (i, 0, 0)),
            pl.BlockSpec((1, 1), lambda i: (0, 0)),
        ],
        out_specs=[
            pl.BlockSpec((1, 1), lambda i: (0, 0)),
            pl.BlockSpec((1, 1), lambda i: (0, 0)),
            pl.BlockSpec((1, 1), lambda i: (0, 0)),
        ],
        out_shape=[
            jax.ShapeDtypeStruct((1, 1), jnp.float32),
            jax.ShapeDtypeStruct((1, 1), jnp.float32),
            jax.ShapeDtypeStruct((1, 1), jnp.float32),
        ],
    )(logits2, klsum)

    return (loss.reshape(()),
            jax.lax.stop_gradient(recon.reshape(())),
            jax.lax.stop_gradient(kl.reshape(())))


# speedup vs baseline: 1.1125x; 1.0279x over previous
import jax
import jax.numpy as jnp
from jax import lax
from jax.experimental import pallas as pl
from jax.experimental.pallas import tpu as pltpu
from jax.experimental.pallas import tpu_sc as plsc

N = 10000
E = 320000
IN_DIM = 128
HID = 64
ZD = 32

NC, NS, L = 2, 16, 16
NW = NC * NS
EP = E // NW
EPD = 2 * E // NW
SUBK = 80
NSUBC = 5
CHE = SUBK * NSUBC
NCH_A = EP // CHE
NCH_D = EPD // CHE
NPAD = 10240
NPT = NPAD // NS
ZROWS = 160

RB = 1000
GRID = N // RB

_MESH = plsc.VectorSubcoreMesh(
    core_axis_name="c", subcore_axis_name="s", num_cores=NC, num_subcores=NS)
_SC_PARAMS = pltpu.CompilerParams(needs_layout_passes=False,
                                  use_tc_tiling_on_sc=False)


def _sc_deg_body(ei_hbm, out_hbm, dstbuf, countbuf):
    cid = lax.axis_index("c")
    sid = lax.axis_index("s")
    wid = sid * NC + cid
    zeros16 = jnp.zeros((L,), jnp.float32)
    ones16 = jnp.ones((L,), jnp.float32)

    def zb(i, c):
        countbuf[pl.ds(i * L, L)] = zeros16
        return c
    lax.fori_loop(0, N // L, zb, 0)

    pltpu.sync_copy(ei_hbm.at[1, pl.ds(wid * EP, EP)], dstbuf)

    def cb(i, c):
        idx = dstbuf[pl.ds(i * L, L)]
        plsc.addupdate_scatter(countbuf, [idx], ones16)
        return c
    lax.fori_loop(0, EP // L, cb, 0)

    for g in range(GRID):
        pltpu.sync_copy(countbuf.at[pl.ds(g * RB, RB)], out_hbm.at[g, wid])


_deg_call = pl.kernel(
    _sc_deg_body,
    out_type=jax.ShapeDtypeStruct((GRID, NW, RB), jnp.float32),
    mesh=_MESH,
    compiler_params=_SC_PARAMS,
    scratch_types=[
        pltpu.VMEM((EP,), jnp.int32),
        pltpu.VMEM((N,), jnp.float32),
    ],
)


def _sc_agg_body(tab_hbm, ei_hbm, out_hbm, sidx, didx, rows, zbuf,
                 acc, gsem, ssem):
    cid = lax.axis_index("c")
    sid = lax.axis_index("s")
    wid = sid * NC + cid
    zeros16 = jnp.zeros((L,), jnp.float32)

    def zb(i, c):
        for j in range(HID // L):
            zbuf[i, pl.ds(j * L, L)] = zeros16
        return c
    lax.fori_loop(0, ZROWS, zb, 0)
    for r in range(NPT // ZROWS):
        pltpu.sync_copy(zbuf, acc.at[pl.ds(sid * NPT + r * ZROWS, ZROWS)])
    plsc.subcore_barrier()

    pltpu.sync_copy(ei_hbm.at[0, pl.ds(wid * EP, EP)], sidx)
    pltpu.sync_copy(ei_hbm.at[1, pl.ds(wid * EP, EP)], didx)

    def start_gathers(c, b):
        for j in range(NSUBC):
            pltpu.async_copy(
                tab_hbm.at[sidx.at[pl.ds(c * CHE + j * SUBK, SUBK)]],
                rows.at[b].at[pl.ds(j * SUBK, SUBK)], gsem.at[b])

    start_gathers(0, 0)
    start_gathers(1, 1)

    def body(c, carry):
        b = lax.rem(c, 2)
        pltpu.make_async_copy(tab_hbm.at[pl.ds(0, CHE)], rows.at[b],
                              gsem.at[b]).wait()
        for j in range(NSUBC):
            pltpu.async_copy(rows.at[b].at[pl.ds(j * SUBK, SUBK)],
                             acc.at[didx.at[pl.ds((c * NSUBC + j) * SUBK, SUBK)]],
                             ssem.at[b], add=True)
        pltpu.make_async_copy(tab_hbm.at[pl.ds(0, CHE)], rows.at[b],
                              ssem.at[b]).wait()

        @pl.when(c + 2 < NCH_A)
        def _():
            start_gathers(c + 2, b)
        return carry
    lax.fori_loop(0, NCH_A, body, 0)

    plsc.subcore_barrier()
    pltpu.sync_copy(acc.at[pl.ds(sid * NPT, NPT)],
                    out_hbm.at[cid, pl.ds(sid * NPT, NPT)])


_agg_call = pl.kernel(
    _sc_agg_body,
    out_type=jax.ShapeDtypeStruct((NC, NPAD, HID), jnp.float32),
    mesh=_MESH,
    compiler_params=_SC_PARAMS,
    scratch_types=[
        pltpu.VMEM((EP,), jnp.int32),
        pltpu.VMEM((EP,), jnp.int32),
        pltpu.VMEM((2, CHE, HID), jnp.float32),
        pltpu.VMEM((ZROWS, HID), jnp.float32),
        pltpu.VMEM_SHARED((NPAD, HID), jnp.float32),
        pltpu.SemaphoreType.DMA((2,)),
        pltpu.SemaphoreType.DMA((2,)),
    ],
)


PTP = 24


def _sc_dec_body(z_hbm, ei_hbm, nei_hbm, out_hbm,
                 sidx, didx, zs, zd, lbuf, ptmp, gsem):
    cid = lax.axis_index("c")
    sid = lax.axis_index("s")
    wid = sid * NC + cid
    iota = lax.iota(jnp.int32, L)
    zeros16 = jnp.zeros((L,), jnp.float32)

    pltpu.sync_copy(ei_hbm.at[0, pl.ds(wid * EP, EP)], sidx.at[pl.ds(0, EP)])
    pltpu.sync_copy(nei_hbm.at[0, pl.ds(wid * EP, EP)], sidx.at[pl.ds(EP, EP)])
    pltpu.sync_copy(ei_hbm.at[1, pl.ds(wid * EP, EP)], didx.at[pl.ds(0, EP)])
    pltpu.sync_copy(nei_hbm.at[1, pl.ds(wid * EP, EP)], didx.at[pl.ds(EP, EP)])

    def start_gathers(c, b):
        for j in range(NSUBC):
            off = c * CHE + j * SUBK
            pltpu.async_copy(z_hbm.at[sidx.at[pl.ds(off, SUBK)]],
                             zs.at[b].at[pl.ds(j * SUBK, SUBK)], gsem.at[b])
            pltpu.async_copy(z_hbm.at[didx.at[pl.ds(off, SUBK)]],
                             zd.at[b].at[pl.ds(j * SUBK, SUBK)], gsem.at[b])

    start_gathers(0, 0)
    start_gathers(1, 1)

    def body(c, carry):
        b = lax.rem(c, 2)
        pltpu.make_async_copy(z_hbm.at[pl.ds(0, CHE)], zs.at[b],
                              gsem.at[b]).wait()
        pltpu.make_async_copy(z_hbm.at[pl.ds(0, CHE)], zd.at[b],
                              gsem.at[b]).wait()

        def gb(g5, cc):
            for sl in range(5):
                g = g5 * 5 + sl
                row0 = g * L
                for k in range(L):
                    a = zs[b, row0 + k, :]
                    d = zd[b, row0 + k, :]
                    pe, po = plsc.unpack(a * d,
                                         format=plsc.PackFormat.INTERLEAVED)
                    ptmp[sl, k, pl.ds(0, L)] = pe + po
                qs = []
                for col in range(L):
                    colv = jnp.full((L,), col, jnp.int32)
                    qs.append(plsc.load_gather(ptmp.at[sl], [iota, colv]))
                while len(qs) > 1:
                    qs = [qs[i] + qs[i + 1] for i in range(0, len(qs) - 1, 2)]                          + ([qs[-1]] if len(qs) % 2 else [])
                lbuf[pl.ds(c * CHE + g * L, L)] = qs[0]
            return cc
        lax.fori_loop(0, (CHE // L) // 5, gb, 0)

        @pl.when(c + 2 < NCH_D)
        def _():
            start_gathers(c + 2, b)
        return carry
    lax.fori_loop(0, NCH_D, body, 0)

    pltpu.sync_copy(lbuf.at[pl.ds(0, EP)], out_hbm.at[0, wid])
    pltpu.sync_copy(lbuf.at[pl.ds(EP, EP)], out_hbm.at[1, wid])


_dec_call = pl.kernel(
    _sc_dec_body,
    out_type=jax.ShapeDtypeStruct((2, NW, EP), jnp.float32),
    mesh=_MESH,
    compiler_params=_SC_PARAMS,
    scratch_types=[
        pltpu.VMEM((EPD,), jnp.int32),
        pltpu.VMEM((EPD,), jnp.int32),
        pltpu.VMEM((2, CHE, ZD), jnp.bfloat16),
        pltpu.VMEM((2, CHE, ZD), jnp.bfloat16),
        pltpu.VMEM((EPD,), jnp.float32),
        pltpu.VMEM((5, L, PTP), jnp.float32),
        pltpu.SemaphoreType.DMA((2,)),
    ],
)


def _tc_prep_body(counts_ref, x_ref, w1_ref, hs_ref, dinv_ref):
    deg = jnp.sum(counts_ref[0], axis=0) + 1.0
    dinv = lax.rsqrt(deg)
    h = jnp.dot(x_ref[...], w1_ref[...], preferred_element_type=jnp.float32)
    hs_ref[...] = h * dinv[:, None]
    dinv_ref[...] = dinv[:, None]


def _tc_h_body(t_ref, hs_ref, dinv_ref, b1_ref, out_ref):
    t = t_ref[0] + t_ref[1] + hs_ref[...]
    dinv = dinv_ref[...]
    h = jnp.maximum(t * dinv + b1_ref[...], 0.0)
    out_ref[...] = h * dinv


def _tc_z_body(t_ref, hs2_ref, dinv_ref, eps_ref, wmu_ref, bmu_ref, wlv_ref,
               blv_ref, z_ref, kl_ref):
    i = pl.program_id(0)
    agg = (t_ref[0] + t_ref[1] + hs2_ref[...]) * dinv_ref[...]
    mu = jnp.dot(agg, wmu_ref[...], preferred_element_type=jnp.float32) + bmu_ref[...]
    lv = jnp.dot(agg, wlv_ref[...], preferred_element_type=jnp.float32) + blv_ref[...]
    z_ref[...] = (mu + eps_ref[...] * jnp.exp(0.5 * lv)).astype(jnp.bfloat16)
    klp = jnp.sum(1.0 + lv - mu * mu - jnp.exp(lv)).reshape(1, 1)

    @pl.when(i == 0)
    def _():
        kl_ref[...] = klp

    @pl.when(i > 0)
    def _():
        kl_ref[...] = kl_ref[...] + klp


def _tc_loss_body(lg_ref, kl_ref, loss_ref, recon_ref, klo_ref):
    i = pl.program_id(0)
    l = lg_ref[0]
    lab = (i == 0).astype(jnp.float32)
    s = jnp.sum(jnp.maximum(l, 0.0) - l * lab
                + jnp.log1p(jnp.exp(-jnp.abs(l)))).reshape(1, 1)

    @pl.when(i == 0)
    def _():
        loss_ref[...] = s

    @pl.when(i == 1)
    def _():
        recon = (loss_ref[...] + s) / (2.0 * E)
        kl = -0.5 * kl_ref[...] / (N * ZD)
        loss_ref[...] = recon + kl
        recon_ref[...] = recon
        klo_ref[...] = kl


def kernel(x, edge_index, neg_edge_index, eps, W1, b1, Wmu, bmu, Wlv, blv):
    ei = edge_index.astype(jnp.int32)
    nei = neg_edge_index.astype(jnp.int32)

    counts = _deg_call(ei)

    hs1, dinv = pl.pallas_call(
        _tc_prep_body,
        grid=(GRID,),
        in_specs=[
            pl.BlockSpec((1, NW, RB), lambda i: (i, 0, 0)),
            pl.BlockSpec((RB, IN_DIM), lambda i: (i, 0)),
            pl.BlockSpec((IN_DIM, HID), lambda i: (0, 0)),
        ],
        out_specs=[
            pl.BlockSpec((RB, HID), lambda i: (i, 0)),
            pl.BlockSpec((RB, 1), lambda i: (i, 0)),
        ],
        out_shape=[
            jax.ShapeDtypeStruct((N, HID), jnp.float32),
            jax.ShapeDtypeStruct((N, 1), jnp.float32),
        ],
    )(counts, x, W1)

    t1 = _agg_call(hs1, ei)

    hs2 = pl.pallas_call(
        _tc_h_body,
        grid=(GRID,),
        in_specs=[
            pl.BlockSpec((NC, RB, HID), lambda i: (0, i, 0)),
            pl.BlockSpec((RB, HID), lambda i: (i, 0)),
            pl.BlockSpec((RB, 1), lambda i: (i, 0)),
            pl.BlockSpec((1, HID), lambda i: (0, 0)),
        ],
        out_specs=pl.BlockSpec((RB, HID), lambda i: (i, 0)),
        out_shape=jax.ShapeDtypeStruct((N, HID), jnp.float32),
    )(t1, hs1, dinv, b1.reshape(1, HID))

    t2 = _agg_call(hs2, ei)

    z, klsum = pl.pallas_call(
        _tc_z_body,
        grid=(GRID,),
        in_specs=[
            pl.BlockSpec((NC, RB, HID), lambda i: (0, i, 0)),
            pl.BlockSpec((RB, HID), lambda i: (i, 0)),
            pl.BlockSpec((RB, 1), lambda i: (i, 0)),
            pl.BlockSpec((RB, ZD), lambda i: (i, 0)),
            pl.BlockSpec((HID, ZD), lambda i: (0, 0)),
            pl.BlockSpec((1, ZD), lambda i: (0, 0)),
            pl.BlockSpec((HID, ZD), lambda i: (0, 0)),
            pl.BlockSpec((1, ZD), lambda i: (0, 0)),
        ],
        out_specs=[
            pl.BlockSpec((RB, ZD), lambda i: (i, 0)),
            pl.BlockSpec((1, 1), lambda i: (0, 0)),
        ],
        out_shape=[
            jax.ShapeDtypeStruct((N, ZD), jnp.bfloat16),
            jax.ShapeDtypeStruct((1, 1), jnp.float32),
        ],
    )(t2, hs2, dinv, eps, Wmu, bmu.reshape(1, ZD), Wlv, blv.reshape(1, ZD))

    logits2 = _dec_call(z, ei, nei)

    loss, recon, kl = pl.pallas_call(
        _tc_loss_body,
        grid=(2,),
        in_specs=[
            pl.BlockSpec((1, NW, EP), lambda i: (i, 0, 0)),
            pl.BlockSpec((1, 1), lambda i: (0, 0)),
        ],
        out_specs=[
            pl.BlockSpec((1, 1), lambda i: (0, 0)),
            pl.BlockSpec((1, 1), lambda i: (0, 0)),
            pl.BlockSpec((1, 1), lambda i: (0, 0)),
        ],
        out_shape=[
            jax.ShapeDtypeStruct((1, 1), jnp.float32),
            jax.ShapeDtypeStruct((1, 1), jnp.float32),
            jax.ShapeDtypeStruct((1, 1), jnp.float32),
        ],
    )(logits2, klsum)

    return (loss.reshape(()),
            jax.lax.stop_gradient(recon.reshape(())),
            jax.lax.stop_gradient(kl.reshape(())))


# speedup vs baseline: 1.1353x; 1.0205x over previous
import jax
import jax.numpy as jnp
from jax import lax
from jax.experimental import pallas as pl
from jax.experimental.pallas import tpu as pltpu
from jax.experimental.pallas import tpu_sc as plsc

N = 10000
E = 320000
IN_DIM = 128
HID = 64
ZD = 32

NC, NS, L = 2, 16, 16
NW = NC * NS
EP = E // NW
EPD = 2 * E // NW
SUBK = 80
NSUBC = 5
CHE = SUBK * NSUBC
NCH_A = EP // CHE
NCH_D = EPD // CHE
NPAD = 10240
NPT = NPAD // NS
ZROWS = 160

RB = 2000
GRID = N // RB

_MESH = plsc.VectorSubcoreMesh(
    core_axis_name="c", subcore_axis_name="s", num_cores=NC, num_subcores=NS)
_SC_PARAMS = pltpu.CompilerParams(needs_layout_passes=False,
                                  use_tc_tiling_on_sc=False)


def _sc_deg_body(ei_hbm, out_hbm, dstbuf, countbuf):
    cid = lax.axis_index("c")
    sid = lax.axis_index("s")
    wid = sid * NC + cid
    zeros16 = jnp.zeros((L,), jnp.float32)
    ones16 = jnp.ones((L,), jnp.float32)

    def zb(i, c):
        countbuf[pl.ds(i * L, L)] = zeros16
        return c
    lax.fori_loop(0, N // L, zb, 0)

    pltpu.sync_copy(ei_hbm.at[1, pl.ds(wid * EP, EP)], dstbuf)

    def cb(i, c):
        idx = dstbuf[pl.ds(i * L, L)]
        plsc.addupdate_scatter(countbuf, [idx], ones16)
        return c
    lax.fori_loop(0, EP // L, cb, 0)

    for g in range(GRID):
        pltpu.sync_copy(countbuf.at[pl.ds(g * RB, RB)], out_hbm.at[g, wid])


_deg_call = pl.kernel(
    _sc_deg_body,
    out_type=jax.ShapeDtypeStruct((GRID, NW, RB), jnp.float32),
    mesh=_MESH,
    compiler_params=_SC_PARAMS,
    scratch_types=[
        pltpu.VMEM((EP,), jnp.int32),
        pltpu.VMEM((N,), jnp.float32),
    ],
)


def _sc_agg_body(tab_hbm, ei_hbm, out_hbm, sidx, didx, rows, zbuf,
                 acc, gsem, ssem):
    cid = lax.axis_index("c")
    sid = lax.axis_index("s")
    wid = sid * NC + cid
    zeros16 = jnp.zeros((L,), jnp.float32)

    def zb(i, c):
        for j in range(HID // L):
            zbuf[i, pl.ds(j * L, L)] = zeros16
        return c
    lax.fori_loop(0, ZROWS, zb, 0)
    for r in range(NPT // ZROWS):
        pltpu.sync_copy(zbuf, acc.at[pl.ds(sid * NPT + r * ZROWS, ZROWS)])
    plsc.subcore_barrier()

    pltpu.sync_copy(ei_hbm.at[0, pl.ds(wid * EP, EP)], sidx)
    pltpu.sync_copy(ei_hbm.at[1, pl.ds(wid * EP, EP)], didx)

    def start_gathers(c, b):
        for j in range(NSUBC):
            pltpu.async_copy(
                tab_hbm.at[sidx.at[pl.ds(c * CHE + j * SUBK, SUBK)]],
                rows.at[b].at[pl.ds(j * SUBK, SUBK)], gsem.at[b])

    start_gathers(0, 0)
    start_gathers(1, 1)

    def body(c, carry):
        b = lax.rem(c, 2)
        pltpu.make_async_copy(tab_hbm.at[pl.ds(0, CHE)], rows.at[b],
                              gsem.at[b]).wait()
        for j in range(NSUBC):
            pltpu.async_copy(rows.at[b].at[pl.ds(j * SUBK, SUBK)],
                             acc.at[didx.at[pl.ds((c * NSUBC + j) * SUBK, SUBK)]],
                             ssem.at[b], add=True)
        pltpu.make_async_copy(tab_hbm.at[pl.ds(0, CHE)], rows.at[b],
                              ssem.at[b]).wait()

        @pl.when(c + 2 < NCH_A)
        def _():
            start_gathers(c + 2, b)
        return carry
    lax.fori_loop(0, NCH_A, body, 0)

    plsc.subcore_barrier()
    pltpu.sync_copy(acc.at[pl.ds(sid * NPT, NPT)],
                    out_hbm.at[cid, pl.ds(sid * NPT, NPT)])


_agg_call = pl.kernel(
    _sc_agg_body,
    out_type=jax.ShapeDtypeStruct((NC, NPAD, HID), jnp.float32),
    mesh=_MESH,
    compiler_params=_SC_PARAMS,
    scratch_types=[
        pltpu.VMEM((EP,), jnp.int32),
        pltpu.VMEM((EP,), jnp.int32),
        pltpu.VMEM((2, CHE, HID), jnp.float32),
        pltpu.VMEM((ZROWS, HID), jnp.float32),
        pltpu.VMEM_SHARED((NPAD, HID), jnp.float32),
        pltpu.SemaphoreType.DMA((2,)),
        pltpu.SemaphoreType.DMA((2,)),
    ],
)


PTP = 24


def _sc_dec_body(z_hbm, ei_hbm, nei_hbm, out_hbm,
                 sidx, didx, zs, zd, lbuf, ptmp, gsem):
    cid = lax.axis_index("c")
    sid = lax.axis_index("s")
    wid = sid * NC + cid
    iota = lax.iota(jnp.int32, L)
    zeros16 = jnp.zeros((L,), jnp.float32)

    pltpu.sync_copy(ei_hbm.at[0, pl.ds(wid * EP, EP)], sidx.at[pl.ds(0, EP)])
    pltpu.sync_copy(nei_hbm.at[0, pl.ds(wid * EP, EP)], sidx.at[pl.ds(EP, EP)])
    pltpu.sync_copy(ei_hbm.at[1, pl.ds(wid * EP, EP)], didx.at[pl.ds(0, EP)])
    pltpu.sync_copy(nei_hbm.at[1, pl.ds(wid * EP, EP)], didx.at[pl.ds(EP, EP)])

    def start_gathers(c, b):
        for j in range(NSUBC):
            off = c * CHE + j * SUBK
            pltpu.async_copy(z_hbm.at[sidx.at[pl.ds(off, SUBK)]],
                             zs.at[b].at[pl.ds(j * SUBK, SUBK)], gsem.at[b])
            pltpu.async_copy(z_hbm.at[didx.at[pl.ds(off, SUBK)]],
                             zd.at[b].at[pl.ds(j * SUBK, SUBK)], gsem.at[b])

    start_gathers(0, 0)
    start_gathers(1, 1)

    def body(c, carry):
        b = lax.rem(c, 2)
        pltpu.make_async_copy(z_hbm.at[pl.ds(0, CHE)], zs.at[b],
                              gsem.at[b]).wait()
        pltpu.make_async_copy(z_hbm.at[pl.ds(0, CHE)], zd.at[b],
                              gsem.at[b]).wait()

        def gb(g5, cc):
            for sl in range(5):
                g = g5 * 5 + sl
                row0 = g * L
                for k in range(L):
                    a = zs[b, row0 + k, :]
                    d = zd[b, row0 + k, :]
                    pe, po = plsc.unpack(a * d,
                                         format=plsc.PackFormat.INTERLEAVED)
                    ptmp[sl, k, pl.ds(0, L)] = pe + po
                qs = []
                for col in range(L):
                    colv = jnp.full((L,), col, jnp.int32)
                    qs.append(plsc.load_gather(ptmp.at[sl], [iota, colv]))
                while len(qs) > 1:
                    qs = [qs[i] + qs[i + 1] for i in range(0, len(qs) - 1, 2)]                          + ([qs[-1]] if len(qs) % 2 else [])
                lbuf[pl.ds(c * CHE + g * L, L)] = qs[0]
            return cc
        lax.fori_loop(0, (CHE // L) // 5, gb, 0)

        @pl.when(c + 2 < NCH_D)
        def _():
            start_gathers(c + 2, b)
        return carry
    lax.fori_loop(0, NCH_D, body, 0)

    pltpu.sync_copy(lbuf.at[pl.ds(0, EP)], out_hbm.at[0, wid])
    pltpu.sync_copy(lbuf.at[pl.ds(EP, EP)], out_hbm.at[1, wid])


_dec_call = pl.kernel(
    _sc_dec_body,
    out_type=jax.ShapeDtypeStruct((2, NW, EP), jnp.float32),
    mesh=_MESH,
    compiler_params=_SC_PARAMS,
    scratch_types=[
        pltpu.VMEM((EPD,), jnp.int32),
        pltpu.VMEM((EPD,), jnp.int32),
        pltpu.VMEM((2, CHE, ZD), jnp.bfloat16),
        pltpu.VMEM((2, CHE, ZD), jnp.bfloat16),
        pltpu.VMEM((EPD,), jnp.float32),
        pltpu.VMEM((5, L, PTP), jnp.float32),
        pltpu.SemaphoreType.DMA((2,)),
    ],
)


def _tc_prep_body(counts_ref, x_ref, w1_ref, hs_ref, dinv_ref):
    deg = jnp.sum(counts_ref[0], axis=0) + 1.0
    dinv = lax.rsqrt(deg)
    h = jnp.dot(x_ref[...], w1_ref[...], preferred_element_type=jnp.float32)
    hs_ref[...] = h * dinv[:, None]
    dinv_ref[...] = dinv[:, None]


def _tc_h_body(t_ref, hs_ref, dinv_ref, b1_ref, out_ref):
    t = t_ref[0] + t_ref[1] + hs_ref[...]
    dinv = dinv_ref[...]
    h = jnp.maximum(t * dinv + b1_ref[...], 0.0)
    out_ref[...] = h * dinv


def _tc_z_body(t_ref, hs2_ref, dinv_ref, eps_ref, wmu_ref, bmu_ref, wlv_ref,
               blv_ref, z_ref, kl_ref):
    i = pl.program_id(0)
    agg = (t_ref[0] + t_ref[1] + hs2_ref[...]) * dinv_ref[...]
    mu = jnp.dot(agg, wmu_ref[...], preferred_element_type=jnp.float32) + bmu_ref[...]
    lv = jnp.dot(agg, wlv_ref[...], preferred_element_type=jnp.float32) + blv_ref[...]
    z_ref[...] = (mu + eps_ref[...] * jnp.exp(0.5 * lv)).astype(jnp.bfloat16)
    klp = jnp.sum(1.0 + lv - mu * mu - jnp.exp(lv)).reshape(1, 1)

    @pl.when(i == 0)
    def _():
        kl_ref[...] = klp

    @pl.when(i > 0)
    def _():
        kl_ref[...] = kl_ref[...] + klp


def _tc_loss_body(lg_ref, kl_ref, loss_ref, recon_ref, klo_ref):
    lp = lg_ref[0]
    ln = lg_ref[1]
    sp = jnp.sum(jnp.maximum(lp, 0.0) - lp + jnp.log1p(jnp.exp(-jnp.abs(lp))))
    sn = jnp.sum(jnp.maximum(ln, 0.0) + jnp.log1p(jnp.exp(-jnp.abs(ln))))
    recon = ((sp + sn) / (2.0 * E)).reshape(1, 1)
    kl = -0.5 * kl_ref[...] / (N * ZD)
    loss_ref[...] = recon + kl
    recon_ref[...] = recon
    klo_ref[...] = kl


def kernel(x, edge_index, neg_edge_index, eps, W1, b1, Wmu, bmu, Wlv, blv):
    ei = edge_index.astype(jnp.int32)
    nei = neg_edge_index.astype(jnp.int32)

    counts = _deg_call(ei)

    hs1, dinv = pl.pallas_call(
        _tc_prep_body,
        grid=(GRID,),
        in_specs=[
            pl.BlockSpec((1, NW, RB), lambda i: (i, 0, 0)),
            pl.BlockSpec((RB, IN_DIM), lambda i: (i, 0)),
            pl.BlockSpec((IN_DIM, HID), lambda i: (0, 0)),
        ],
        out_specs=[
            pl.BlockSpec((RB, HID), lambda i: (i, 0)),
            pl.BlockSpec((RB, 1), lambda i: (i, 0)),
        ],
        out_shape=[
            jax.ShapeDtypeStruct((N, HID), jnp.float32),
            jax.ShapeDtypeStruct((N, 1), jnp.float32),
        ],
    )(counts, x, W1)

    t1 = _agg_call(hs1, ei)

    hs2 = pl.pallas_call(
        _tc_h_body,
        grid=(GRID,),
        in_specs=[
            pl.BlockSpec((NC, RB, HID), lambda i: (0, i, 0)),
            pl.BlockSpec((RB, HID), lambda i: (i, 0)),
            pl.BlockSpec((RB, 1), lambda i: (i, 0)),
            pl.BlockSpec((1, HID), lambda i: (0, 0)),
        ],
        out_specs=pl.BlockSpec((RB, HID), lambda i: (i, 0)),
        out_shape=jax.ShapeDtypeStruct((N, HID), jnp.float32),
    )(t1, hs1, dinv, b1.reshape(1, HID))

    t2 = _agg_call(hs2, ei)

    z, klsum = pl.pallas_call(
        _tc_z_body,
        grid=(GRID,),
        in_specs=[
            pl.BlockSpec((NC, RB, HID), lambda i: (0, i, 0)),
            pl.BlockSpec((RB, HID), lambda i: (i, 0)),
            pl.BlockSpec((RB, 1), lambda i: (i, 0)),
            pl.BlockSpec((RB, ZD), lambda i: (i, 0)),
            pl.BlockSpec((HID, ZD), lambda i: (0, 0)),
            pl.BlockSpec((1, ZD), lambda i: (0, 0)),
            pl.BlockSpec((HID, ZD), lambda i: (0, 0)),
            pl.BlockSpec((1, ZD), lambda i: (0, 0)),
        ],
        out_specs=[
            pl.BlockSpec((RB, ZD), lambda i: (i, 0)),
            pl.BlockSpec((1, 1), lambda i: (0, 0)),
        ],
        out_shape=[
            jax.ShapeDtypeStruct((N, ZD), jnp.bfloat16),
            jax.ShapeDtypeStruct((1, 1), jnp.float32),
        ],
    )(t2, hs2, dinv, eps, Wmu, bmu.reshape(1, ZD), Wlv, blv.reshape(1, ZD))

    logits2 = _dec_call(z, ei, nei)

    loss, recon, kl = pl.pallas_call(
        _tc_loss_body,
        in_specs=[
            pl.BlockSpec((2, NW, EP), lambda: (0, 0, 0)),
            pl.BlockSpec((1, 1), lambda: (0, 0)),
        ],
        out_specs=[
            pl.BlockSpec((1, 1), lambda: (0, 0)),
            pl.BlockSpec((1, 1), lambda: (0, 0)),
            pl.BlockSpec((1, 1), lambda: (0, 0)),
        ],
        out_shape=[
            jax.ShapeDtypeStruct((1, 1), jnp.float32),
            jax.ShapeDtypeStruct((1, 1), jnp.float32),
            jax.ShapeDtypeStruct((1, 1), jnp.float32),
        ],
    )(logits2, klsum)

    return (loss.reshape(()),
            jax.lax.stop_gradient(recon.reshape(())),
            jax.lax.stop_gradient(kl.reshape(())))
